# baseline (speedup 1.0000x reference)
import functools

import jax
import jax.numpy as jnp
import numpy as np
from jax import lax
from jax.experimental import pallas as pl
from jax.experimental.pallas import tpu as pltpu
from jax.experimental.pallas import tpu_sc as plsc

_EPS = float(np.finfo(np.float32).eps)

NW = 32
TB = 288


def _router_body(x_ref, lnw_ref, wg_ref, bg_ref, xn_ref, dst_ref, gw_ref,
                 meta_ref, *, n_experts, nb):
    E = n_experts
    S = x_ref.shape[0]
    x = x_ref[...]
    ms = jnp.mean(x * x, axis=-1, keepdims=True)
    xn = x * jax.lax.rsqrt(ms + _EPS) * lnw_ref[...]
    xn_ref[...] = xn
    logits = jax.lax.dot_general(
        xn, wg_ref[...], (((1,), (1,)), ((), ())),
        preferred_element_type=jnp.float32) + bg_ref[...]
    iota = jax.lax.broadcasted_iota(jnp.int32, logits.shape, 1)
    m1 = jnp.max(logits, axis=-1, keepdims=True)
    i1 = jnp.min(jnp.where(logits == m1, iota, E), axis=-1, keepdims=True)
    l2 = jnp.where(iota == i1, -jnp.inf, logits)
    m2 = jnp.max(l2, axis=-1, keepdims=True)
    i2 = jnp.min(jnp.where(l2 == m2, iota, E), axis=-1, keepdims=True)
    p = jnp.exp(m2 - m1)
    g1 = 1.0 / (1.0 + p)
    gw_ref[...] = jnp.concatenate([g1, p * g1], axis=1)

    oh1 = (iota == i1).astype(jnp.float32)
    oh2 = (iota == i2).astype(jnp.float32)
    ohs = oh1 + oh2
    ri = jax.lax.broadcasted_iota(jnp.int32, (S, S), 0)
    ci = jax.lax.broadcasted_iota(jnp.int32, (S, S), 1)
    tri = jnp.where(ri > ci, 1.0, 0.0)
    cum_ex = jax.lax.dot_general(tri, ohs, (((1,), (0,)), ((), ())),
                                 preferred_element_type=jnp.float32)

    offs = []
    ends_blk = []
    run = jnp.int32(0)
    for e in range(E):
        g_e = jnp.sum(jnp.where(iota == e, ohs, 0.0)).astype(jnp.int32)
        offs.append(run)
        nblk_e = (g_e + (TB - 1)) // TB
        run = run + nblk_e * TB
        ends_blk.append(run // TB)
    used = run // TB

    off_vec = jnp.zeros((1, E), jnp.float32)
    for e in range(E):
        off_vec = jnp.where(iota[:1] == e, offs[e].astype(jnp.float32),
                            off_vec)
    base = off_vec + cum_ex
    d1 = jnp.sum(oh1 * base, axis=1, keepdims=True)
    d2 = jnp.sum(oh2 * (base + oh1), axis=1, keepdims=True)
    dst_ref[...] = jnp.concatenate([d1, d2], axis=1).astype(jnp.int32)

    del used, nb
    iota32 = jax.lax.broadcasted_iota(jnp.int32, (1, 32), 1)
    metav = jnp.zeros((1, 32), jnp.int32)
    for e in range(E):
        metav = jnp.where(iota32 == e + 1, ends_blk[e], metav)
    meta_ref[...] = metav


def _make_dispatch(P, S, D, PMAX):
    PPW = P // NW
    NCH = PPW // 32
    mesh = plsc.VectorSubcoreMesh(core_axis_name="c", subcore_axis_name="s")

    @functools.partial(
        pl.kernel, mesh=mesh,
        out_type=jax.ShapeDtypeStruct((PMAX, D), jnp.float32),
        scratch_types=[
            pltpu.VMEM((NCH, 32), jnp.int32),
            pltpu.VMEM((32,), jnp.int32),
            pltpu.VMEM((32, D), jnp.float32),
            pltpu.SemaphoreType.DMA,
            pltpu.SemaphoreType.DMA,
        ],
    )
    def dispatch_kernel(tok_hbm, dst_hbm, xn_hbm, xs_hbm, dstb, tokb, rows,
                        sem1, sem2):
        w = lax.axis_index("s") * 2 + lax.axis_index("c")
        for c in range(NCH):
            pltpu.sync_copy(dst_hbm.at[pl.ds(w * PPW + c * 32, 32)],
                            dstb.at[c])
        for c in range(NCH):
            base = w * PPW + c * 32
            pltpu.sync_copy(tok_hbm.at[pl.ds(base, 32)], tokb)
            pltpu.async_copy(xn_hbm.at[tokb], rows, sem1).wait()
            pltpu.async_copy(rows, xs_hbm.at[dstb.at[c]], sem2).wait()

    return dispatch_kernel


def _ffn_body(m_ref, xs_ref, w1a_ref, w1ab_ref, w1b_ref, w1bb_ref, w2_ref,
              w2b_ref, ys_ref):
    e = pl.program_id(0)
    i = pl.program_id(1)

    @pl.when(i < m_ref[e + 1] - m_ref[e])
    def _():
        nsub = 2
        rb = xs_ref.shape[0] // nsub
        for r in range(nsub):
            xs = xs_ref[pl.ds(r * rb, rb), :]
            a = jax.lax.dot_general(
                xs, w1a_ref[0], (((1,), (1,)), ((), ())),
                preferred_element_type=jnp.float32) + w1ab_ref[0]
            b = jax.lax.dot_general(
                xs, w1b_ref[0], (((1,), (1,)), ((), ())),
                preferred_element_type=jnp.float32) + w1bb_ref[0]
            h = (a / (1.0 + jnp.exp(-a))) * b
            y = jax.lax.dot_general(
                h, w2_ref[0], (((1,), (1,)), ((), ())),
                preferred_element_type=jnp.float32)
            ys_ref[pl.ds(r * rb, rb), :] = y + w2b_ref[0]


def _make_combine(P, S, D, PMAX, TPW):
    mesh = plsc.VectorSubcoreMesh(core_axis_name="c", subcore_axis_name="s")
    NCH = TPW // 16

    @functools.partial(
        pl.kernel, mesh=mesh,
        out_type=jax.ShapeDtypeStruct((S, D), jnp.float32),
        scratch_types=[
            pltpu.VMEM((NCH, 32), jnp.int32),
            pltpu.VMEM((2 * TPW,), jnp.float32),
            pltpu.VMEM((32, D), jnp.float32),
            pltpu.VMEM((16, D), jnp.float32),
            pltpu.SemaphoreType.DMA,
        ],
    )
    def combine_kernel(ys_hbm, dst_hbm, gw_hbm, out_hbm, dbuf, gbuf, rows,
                       obuf, sem):
        w = lax.axis_index("s") * 2 + lax.axis_index("c")
        for c in range(NCH):
            pltpu.sync_copy(dst_hbm.at[pl.ds(w * 2 * TPW + c * 32, 32)],
                            dbuf.at[c])
        pltpu.sync_copy(gw_hbm.at[pl.ds(w * 2 * TPW, 2 * TPW)], gbuf)
        for c in range(NCH):
            pltpu.async_copy(ys_hbm.at[dbuf.at[c]], rows, sem).wait()
            gv0 = gbuf[pl.ds(c * 32, 16)]
            gv1 = gbuf[pl.ds(c * 32 + 16, 16)]
            ws = []
            for t in range(16):
                gv = gv0 if t < 8 else gv1
                ws.append((gv[(2 * t) % 16], gv[(2 * t + 1) % 16]))

            def body(d, _):
                off = pl.multiple_of(d * 16, 16)
                for t in range(16):
                    w0, w1 = ws[t]
                    obuf[t, pl.ds(off, 16)] = (
                        w0 * rows[2 * t, pl.ds(off, 16)]
                        + w1 * rows[2 * t + 1, pl.ds(off, 16)])
                return 0

            lax.fori_loop(0, D // 16, body, 0)
            pltpu.sync_copy(obuf, out_hbm.at[pl.ds(w * TPW + c * 16, 16)])

    return combine_kernel


def kernel(x, ln_w, Wg, bg, w1a_W, w1a_b, w1b_W, w1b_b, w2_W, w2_b):
    B, S, D = x.shape
    E, F = w1a_W.shape[0], w1a_W.shape[1]
    K = 2
    P = S * K
    NB = P // TB + E + 1
    PMAX = NB * TB
    TPW = S // NW
    xf = x.reshape(S, D)

    xn, dst2, gw2, meta2 = pl.pallas_call(
        functools.partial(_router_body, n_experts=E, nb=NB),
        grid=(1,),
        in_specs=[
            pl.BlockSpec((S, D), lambda i: (0, 0)),
            pl.BlockSpec((1, D), lambda i: (0, 0)),
            pl.BlockSpec((E, D), lambda i: (0, 0)),
            pl.BlockSpec((1, E), lambda i: (0, 0)),
        ],
        out_specs=[
            pl.BlockSpec((S, D), lambda i: (0, 0)),
            pl.BlockSpec((S, K), lambda i: (0, 0)),
            pl.BlockSpec((S, K), lambda i: (0, 0)),
            pl.BlockSpec((1, 32), lambda i: (0, 0)),
        ],
        out_shape=[
            jax.ShapeDtypeStruct((S, D), jnp.float32),
            jax.ShapeDtypeStruct((S, K), jnp.int32),
            jax.ShapeDtypeStruct((S, K), jnp.float32),
            jax.ShapeDtypeStruct((1, 32), jnp.int32),
        ],
        compiler_params=pltpu.CompilerParams(
            vmem_limit_bytes=128 * 1024 * 1024,
        ),
    )(xf, ln_w.reshape(1, D), Wg, bg.reshape(1, E))

    dst = dst2.reshape(P)
    gws = gw2.reshape(P)
    meta = meta2.reshape(32)

    tok = jnp.arange(P, dtype=jnp.int32) // 2
    xs = _make_dispatch(P, S, D, PMAX)(tok, dst, xn)

    MAXBLK = (S + TB - 1) // TB

    def _rows(e, i, m):
        return (jnp.minimum(m[e] + i, jnp.maximum(m[e + 1] - 1, m[e])), 0)

    ys = pl.pallas_call(
        _ffn_body,
        grid_spec=pltpu.PrefetchScalarGridSpec(
            num_scalar_prefetch=1,
            grid=(E, MAXBLK),
            in_specs=[
                pl.BlockSpec((TB, D), _rows),
                pl.BlockSpec((1, F, D), lambda e, i, m: (e, 0, 0)),
                pl.BlockSpec((1, 1, F), lambda e, i, m: (e, 0, 0)),
                pl.BlockSpec((1, F, D), lambda e, i, m: (e, 0, 0)),
                pl.BlockSpec((1, 1, F), lambda e, i, m: (e, 0, 0)),
                pl.BlockSpec((1, D, F), lambda e, i, m: (e, 0, 0)),
                pl.BlockSpec((1, 1, D), lambda e, i, m: (e, 0, 0)),
            ],
            out_specs=pl.BlockSpec((TB, D), _rows),
        ),
        out_shape=jax.ShapeDtypeStruct((PMAX, D), jnp.float32),
        compiler_params=pltpu.CompilerParams(
            dimension_semantics=("arbitrary", "arbitrary"),
            vmem_limit_bytes=128 * 1024 * 1024,
        ),
    )(meta, xs, w1a_W, w1a_b.reshape(E, 1, F), w1b_W,
      w1b_b.reshape(E, 1, F), w2_W, w2_b.reshape(E, 1, D))

    out = _make_combine(P, S, D, PMAX, TPW)(ys, dst, gws)
    return out.reshape(B, S, D)


# speedup vs baseline: 1.5196x; 1.5196x over previous
import functools

import jax
import jax.numpy as jnp
import numpy as np
from jax import lax
from jax.experimental import pallas as pl
from jax.experimental.pallas import tpu as pltpu
from jax.experimental.pallas import tpu_sc as plsc

_EPS = float(np.finfo(np.float32).eps)

NW = 32
TB = 288


def _router_body(x_ref, lnw_ref, wg_ref, bg_ref, xn_ref, dst_ref, gw_ref,
                 meta_ref, *, n_experts, nb):
    E = n_experts
    S = x_ref.shape[0]
    x = x_ref[...]
    ms = jnp.mean(x * x, axis=-1, keepdims=True)
    xn = x * jax.lax.rsqrt(ms + _EPS) * lnw_ref[...]
    xn_ref[...] = xn
    logits = jax.lax.dot_general(
        xn, wg_ref[...], (((1,), (1,)), ((), ())),
        preferred_element_type=jnp.float32) + bg_ref[...]
    iota = jax.lax.broadcasted_iota(jnp.int32, logits.shape, 1)
    m1 = jnp.max(logits, axis=-1, keepdims=True)
    i1 = jnp.min(jnp.where(logits == m1, iota, E), axis=-1, keepdims=True)
    l2 = jnp.where(iota == i1, -jnp.inf, logits)
    m2 = jnp.max(l2, axis=-1, keepdims=True)
    i2 = jnp.min(jnp.where(l2 == m2, iota, E), axis=-1, keepdims=True)
    p = jnp.exp(m2 - m1)
    g1 = 1.0 / (1.0 + p)
    gw_ref[...] = jnp.concatenate([g1, p * g1], axis=1)

    oh1 = (iota == i1).astype(jnp.float32)
    oh2 = (iota == i2).astype(jnp.float32)
    ohs = oh1 + oh2
    ri = jax.lax.broadcasted_iota(jnp.int32, (S, S), 0)
    ci = jax.lax.broadcasted_iota(jnp.int32, (S, S), 1)
    tri = jnp.where(ri > ci, 1.0, 0.0)
    cum_ex = jax.lax.dot_general(tri, ohs, (((1,), (0,)), ((), ())),
                                 preferred_element_type=jnp.float32)

    offs = []
    ends_blk = []
    starts_blk = []
    present = []
    run = jnp.int32(0)
    for e in range(E):
        g_e = jnp.sum(jnp.where(iota == e, ohs, 0.0)).astype(jnp.int32)
        offs.append(run)
        present.append(g_e > 0)
        starts_blk.append(run // TB)
        nblk_e = (g_e + (TB - 1)) // TB
        run = run + nblk_e * TB
        ends_blk.append(run // TB)
    used = run // TB

    off_vec = jnp.zeros((1, E), jnp.float32)
    for e in range(E):
        off_vec = jnp.where(iota[:1] == e, offs[e].astype(jnp.float32),
                            off_vec)
    base = off_vec + cum_ex
    d1 = jnp.sum(oh1 * base, axis=1, keepdims=True)
    d2 = jnp.sum(oh2 * (base + oh1), axis=1, keepdims=True)
    dst_ref[...] = jnp.concatenate([d1, d2], axis=1).astype(jnp.int32)

    iota32 = jax.lax.broadcasted_iota(jnp.int32, (1, 32), 1)
    bid = jnp.zeros((1, 32), jnp.int32)
    rank = jnp.zeros((1, 32), jnp.int32)
    bid_last = jnp.int32(0)
    for e in range(E):
        bid = bid + jnp.where(iota32 >= ends_blk[e], 1, 0)
        bid_last = bid_last + jnp.where(ends_blk[e] <= used - 1, 1, 0)
        rank = rank + jnp.where((iota32 >= starts_blk[e]) & present[e], 1, 0)
    bid = jnp.where(iota32 < used, bid, bid_last)
    slot = jax.lax.rem(rank - 1, 2)
    meta_ref[...] = jnp.where(iota32 == nb, used, bid + 8 * slot)


def _make_dispatch(P, S, D, PMAX):
    PPW = P // NW
    NCH = PPW // 32
    mesh = plsc.VectorSubcoreMesh(core_axis_name="c", subcore_axis_name="s")

    @functools.partial(
        pl.kernel, mesh=mesh,
        out_type=jax.ShapeDtypeStruct((PMAX, D), jnp.float32),
        scratch_types=[
            pltpu.VMEM((NCH, 32), jnp.int32),
            pltpu.VMEM((32,), jnp.int32),
            pltpu.VMEM((32, D), jnp.float32),
            pltpu.SemaphoreType.DMA,
            pltpu.SemaphoreType.DMA,
        ],
    )
    def dispatch_kernel(tok_hbm, dst_hbm, xn_hbm, xs_hbm, dstb, tokb, rows,
                        sem1, sem2):
        w = lax.axis_index("s") * 2 + lax.axis_index("c")
        for c in range(NCH):
            pltpu.sync_copy(dst_hbm.at[pl.ds(w * PPW + c * 32, 32)],
                            dstb.at[c])
        for c in range(NCH):
            base = w * PPW + c * 32
            pltpu.sync_copy(tok_hbm.at[pl.ds(base, 32)], tokb)
            pltpu.async_copy(xn_hbm.at[tokb], rows, sem1).wait()
            pltpu.async_copy(rows, xs_hbm.at[dstb.at[c]], sem2).wait()

    return dispatch_kernel


def _ffn_body(m_ref, xs_ref, w1a_hbm, w1ab_ref, w1b_hbm, w1bb_ref, w2_hbm,
              w2b_ref, ys_ref, w1a_v, w1b_v, w2_v, sems, *, nb):
    j = pl.program_id(0)
    mj = m_ref[j]
    e = jax.lax.rem(mj, 8)
    slot = jax.lax.div(mj, 8)
    used = m_ref[nb]
    prev = m_ref[jnp.maximum(j - 1, 0)]
    nxt = m_ref[jnp.minimum(j + 1, nb - 1)]

    def _start(ee, ss):
        pltpu.make_async_copy(w1a_hbm.at[ee], w1a_v.at[ss],
                              sems.at[ss, 0]).start()
        pltpu.make_async_copy(w1b_hbm.at[ee], w1b_v.at[ss],
                              sems.at[ss, 1]).start()
        pltpu.make_async_copy(w2_hbm.at[ee], w2_v.at[ss],
                              sems.at[ss, 2]).start()

    @pl.when(j == 0)
    def _prime():
        _start(e, slot)

    @pl.when(nxt != mj)
    def _prefetch():
        _start(jax.lax.rem(nxt, 8), jax.lax.div(nxt, 8))

    @pl.when((j == 0) | (prev != mj))
    def _wait():
        pltpu.make_async_copy(w1a_hbm.at[e], w1a_v.at[slot],
                              sems.at[slot, 0]).wait()
        pltpu.make_async_copy(w1b_hbm.at[e], w1b_v.at[slot],
                              sems.at[slot, 1]).wait()
        pltpu.make_async_copy(w2_hbm.at[e], w2_v.at[slot],
                              sems.at[slot, 2]).wait()

    @pl.when(j < used)
    def _():
        xs = xs_ref[...]
        a = jax.lax.dot_general(
            xs, w1a_v[slot], (((1,), (1,)), ((), ())),
            preferred_element_type=jnp.float32) + w1ab_ref[0]
        b = jax.lax.dot_general(
            xs, w1b_v[slot], (((1,), (1,)), ((), ())),
            preferred_element_type=jnp.float32) + w1bb_ref[0]
        h = (a / (1.0 + jnp.exp(-a))) * b
        y = jax.lax.dot_general(
            h, w2_v[slot], (((1,), (1,)), ((), ())),
            preferred_element_type=jnp.float32)
        ys_ref[...] = y + w2b_ref[0]


def _make_combine(P, S, D, PMAX, TPW):
    mesh = plsc.VectorSubcoreMesh(core_axis_name="c", subcore_axis_name="s")
    NCH = TPW // 16

    @functools.partial(
        pl.kernel, mesh=mesh,
        out_type=jax.ShapeDtypeStruct((S, D), jnp.float32),
        scratch_types=[
            pltpu.VMEM((NCH, 32), jnp.int32),
            pltpu.VMEM((2 * TPW,), jnp.float32),
            pltpu.VMEM((32, D), jnp.float32),
            pltpu.VMEM((16, D), jnp.float32),
            pltpu.SemaphoreType.DMA,
        ],
    )
    def combine_kernel(ys_hbm, dst_hbm, gw_hbm, out_hbm, dbuf, gbuf, rows,
                       obuf, sem):
        w = lax.axis_index("s") * 2 + lax.axis_index("c")
        for c in range(NCH):
            pltpu.sync_copy(dst_hbm.at[pl.ds(w * 2 * TPW + c * 32, 32)],
                            dbuf.at[c])
        pltpu.sync_copy(gw_hbm.at[pl.ds(w * 2 * TPW, 2 * TPW)], gbuf)
        for c in range(NCH):
            pltpu.async_copy(ys_hbm.at[dbuf.at[c]], rows, sem).wait()
            gv0 = gbuf[pl.ds(c * 32, 16)]
            gv1 = gbuf[pl.ds(c * 32 + 16, 16)]
            ws = []
            for t in range(16):
                gv = gv0 if t < 8 else gv1
                ws.append((gv[(2 * t) % 16], gv[(2 * t + 1) % 16]))

            def body(d, _):
                off = pl.multiple_of(d * 16, 16)
                for t in range(16):
                    w0, w1 = ws[t]
                    obuf[t, pl.ds(off, 16)] = (
                        w0 * rows[2 * t, pl.ds(off, 16)]
                        + w1 * rows[2 * t + 1, pl.ds(off, 16)])
                return 0

            lax.fori_loop(0, D // 16, body, 0)
            pltpu.sync_copy(obuf, out_hbm.at[pl.ds(w * TPW + c * 16, 16)])

    return combine_kernel


def kernel(x, ln_w, Wg, bg, w1a_W, w1a_b, w1b_W, w1b_b, w2_W, w2_b):
    B, S, D = x.shape
    E, F = w1a_W.shape[0], w1a_W.shape[1]
    K = 2
    P = S * K
    NB = P // TB + E + 1
    PMAX = NB * TB
    TPW = S // NW
    xf = x.reshape(S, D)

    xn, dst2, gw2, meta2 = pl.pallas_call(
        functools.partial(_router_body, n_experts=E, nb=NB),
        grid=(1,),
        in_specs=[
            pl.BlockSpec((S, D), lambda i: (0, 0)),
            pl.BlockSpec((1, D), lambda i: (0, 0)),
            pl.BlockSpec((E, D), lambda i: (0, 0)),
            pl.BlockSpec((1, E), lambda i: (0, 0)),
        ],
        out_specs=[
            pl.BlockSpec((S, D), lambda i: (0, 0)),
            pl.BlockSpec((S, K), lambda i: (0, 0)),
            pl.BlockSpec((S, K), lambda i: (0, 0)),
            pl.BlockSpec((1, 32), lambda i: (0, 0)),
        ],
        out_shape=[
            jax.ShapeDtypeStruct((S, D), jnp.float32),
            jax.ShapeDtypeStruct((S, K), jnp.int32),
            jax.ShapeDtypeStruct((S, K), jnp.float32),
            jax.ShapeDtypeStruct((1, 32), jnp.int32),
        ],
        compiler_params=pltpu.CompilerParams(
            vmem_limit_bytes=128 * 1024 * 1024,
        ),
    )(xf, ln_w.reshape(1, D), Wg, bg.reshape(1, E))

    dst = dst2.reshape(P)
    gws = gw2.reshape(P)
    meta = meta2.reshape(32)

    tok = jnp.arange(P, dtype=jnp.int32) // 2
    xs = _make_dispatch(P, S, D, PMAX)(tok, dst, xn)

    ys = pl.pallas_call(
        functools.partial(_ffn_body, nb=NB),
        grid_spec=pltpu.PrefetchScalarGridSpec(
            num_scalar_prefetch=1,
            grid=(NB,),
            in_specs=[
                pl.BlockSpec((TB, D), lambda j, m: (j, 0)),
                pl.BlockSpec(memory_space=pl.ANY),
                pl.BlockSpec((1, 1, F), lambda j, m: (jax.lax.rem(m[j], 8), 0, 0)),
                pl.BlockSpec(memory_space=pl.ANY),
                pl.BlockSpec((1, 1, F), lambda j, m: (jax.lax.rem(m[j], 8), 0, 0)),
                pl.BlockSpec(memory_space=pl.ANY),
                pl.BlockSpec((1, 1, D), lambda j, m: (jax.lax.rem(m[j], 8), 0, 0)),
            ],
            out_specs=pl.BlockSpec((TB, D), lambda j, m: (j, 0)),
            scratch_shapes=[
                pltpu.VMEM((2, F, D), jnp.float32),
                pltpu.VMEM((2, F, D), jnp.float32),
                pltpu.VMEM((2, D, F), jnp.float32),
                pltpu.SemaphoreType.DMA((2, 3)),
            ],
        ),
        out_shape=jax.ShapeDtypeStruct((PMAX, D), jnp.float32),
        compiler_params=pltpu.CompilerParams(
            dimension_semantics=("arbitrary",),
            vmem_limit_bytes=128 * 1024 * 1024,
        ),
    )(meta, xs, w1a_W, w1a_b.reshape(E, 1, F), w1b_W,
      w1b_b.reshape(E, 1, F), w2_W, w2_b.reshape(E, 1, D))

    out = _make_combine(P, S, D, PMAX, TPW)(ys, dst, gws)
    return out.reshape(B, S, D)


# speedup vs baseline: 1.6106x; 1.0599x over previous
import functools

import jax
import jax.numpy as jnp
import numpy as np
from jax import lax
from jax.experimental import pallas as pl
from jax.experimental.pallas import tpu as pltpu
from jax.experimental.pallas import tpu_sc as plsc

_EPS = float(np.finfo(np.float32).eps)

NW = 32
TB = 512


def _router_body(x_ref, lnw_ref, wg_ref, bg_ref, xn_ref, dst_ref, gw_ref,
                 meta_ref, *, n_experts, nb):
    E = n_experts
    S = x_ref.shape[0]
    x = x_ref[...]
    ms = jnp.mean(x * x, axis=-1, keepdims=True)
    xn = x * jax.lax.rsqrt(ms + _EPS) * lnw_ref[...]
    xn_ref[...] = xn
    logits = jax.lax.dot_general(
        xn, wg_ref[...], (((1,), (1,)), ((), ())),
        preferred_element_type=jnp.float32) + bg_ref[...]
    iota = jax.lax.broadcasted_iota(jnp.int32, logits.shape, 1)
    m1 = jnp.max(logits, axis=-1, keepdims=True)
    i1 = jnp.min(jnp.where(logits == m1, iota, E), axis=-1, keepdims=True)
    l2 = jnp.where(iota == i1, -jnp.inf, logits)
    m2 = jnp.max(l2, axis=-1, keepdims=True)
    i2 = jnp.min(jnp.where(l2 == m2, iota, E), axis=-1, keepdims=True)
    p = jnp.exp(m2 - m1)
    g1 = 1.0 / (1.0 + p)
    gw_ref[...] = jnp.concatenate([g1, p * g1], axis=1)

    oh1 = (iota == i1).astype(jnp.float32)
    oh2 = (iota == i2).astype(jnp.float32)
    ohs = oh1 + oh2
    ri = jax.lax.broadcasted_iota(jnp.int32, (S, S), 0)
    ci = jax.lax.broadcasted_iota(jnp.int32, (S, S), 1)
    tri = jnp.where(ri > ci, 1.0, 0.0)
    cum_ex = jax.lax.dot_general(tri, ohs, (((1,), (0,)), ((), ())),
                                 preferred_element_type=jnp.float32)

    offs = []
    ends_blk = []
    starts_blk = []
    present = []
    run = jnp.int32(0)
    for e in range(E):
        g_e = jnp.sum(jnp.where(iota == e, ohs, 0.0)).astype(jnp.int32)
        offs.append(run)
        present.append(g_e > 0)
        starts_blk.append(run // TB)
        nblk_e = (g_e + (TB - 1)) // TB
        run = run + nblk_e * TB
        ends_blk.append(run // TB)
    used = run // TB

    off_vec = jnp.zeros((1, E), jnp.float32)
    for e in range(E):
        off_vec = jnp.where(iota[:1] == e, offs[e].astype(jnp.float32),
                            off_vec)
    base = off_vec + cum_ex
    d1 = jnp.sum(oh1 * base, axis=1, keepdims=True)
    d2 = jnp.sum(oh2 * (base + oh1), axis=1, keepdims=True)
    dst_ref[...] = jnp.concatenate([d1, d2], axis=1).astype(jnp.int32)

    iota32 = jax.lax.broadcasted_iota(jnp.int32, (1, 32), 1)
    bid = jnp.zeros((1, 32), jnp.int32)
    rank = jnp.zeros((1, 32), jnp.int32)
    bid_last = jnp.int32(0)
    for e in range(E):
        bid = bid + jnp.where(iota32 >= ends_blk[e], 1, 0)
        bid_last = bid_last + jnp.where(ends_blk[e] <= used - 1, 1, 0)
        rank = rank + jnp.where((iota32 >= starts_blk[e]) & present[e], 1, 0)
    bid = jnp.where(iota32 < used, bid, bid_last)
    slot = jax.lax.rem(rank - 1, 2)
    meta_ref[...] = jnp.where(iota32 == nb, used, bid + 8 * slot)


def _make_dispatch(P, S, D, PMAX):
    PPW = P // NW
    NCH = PPW // 32
    mesh = plsc.VectorSubcoreMesh(core_axis_name="c", subcore_axis_name="s")

    @functools.partial(
        pl.kernel, mesh=mesh,
        out_type=jax.ShapeDtypeStruct((PMAX, D), jnp.float32),
        scratch_types=[
            pltpu.VMEM((NCH, 32), jnp.int32),
            pltpu.VMEM((32,), jnp.int32),
            pltpu.VMEM((32, D), jnp.float32),
            pltpu.SemaphoreType.DMA,
            pltpu.SemaphoreType.DMA,
        ],
    )
    def dispatch_kernel(tok_hbm, dst_hbm, xn_hbm, xs_hbm, dstb, tokb, rows,
                        sem1, sem2):
        w = lax.axis_index("s") * 2 + lax.axis_index("c")
        for c in range(NCH):
            pltpu.sync_copy(dst_hbm.at[pl.ds(w * PPW + c * 32, 32)],
                            dstb.at[c])
        for c in range(NCH):
            base = w * PPW + c * 32
            pltpu.sync_copy(tok_hbm.at[pl.ds(base, 32)], tokb)
            pltpu.async_copy(xn_hbm.at[tokb], rows, sem1).wait()
            pltpu.async_copy(rows, xs_hbm.at[dstb.at[c]], sem2).wait()

    return dispatch_kernel


def _ffn_body(m_ref, xs_ref, w1a_hbm, w1ab_ref, w1b_hbm, w1bb_ref, w2_hbm,
              w2b_ref, ys_ref, w1a_v, w1b_v, w2_v, sems, *, nb):
    j = pl.program_id(0)
    mj = m_ref[j]
    e = jax.lax.rem(mj, 8)
    slot = jax.lax.div(mj, 8)
    used = m_ref[nb]
    prev = m_ref[jnp.maximum(j - 1, 0)]
    nxt = m_ref[jnp.minimum(j + 1, nb - 1)]

    def _start(ee, ss):
        pltpu.make_async_copy(w1a_hbm.at[ee], w1a_v.at[ss],
                              sems.at[ss, 0]).start()
        pltpu.make_async_copy(w1b_hbm.at[ee], w1b_v.at[ss],
                              sems.at[ss, 1]).start()
        pltpu.make_async_copy(w2_hbm.at[ee], w2_v.at[ss],
                              sems.at[ss, 2]).start()

    @pl.when(j == 0)
    def _prime():
        _start(e, slot)

    @pl.when(nxt != mj)
    def _prefetch():
        _start(jax.lax.rem(nxt, 8), jax.lax.div(nxt, 8))

    @pl.when((j == 0) | (prev != mj))
    def _wait():
        pltpu.make_async_copy(w1a_hbm.at[e], w1a_v.at[slot],
                              sems.at[slot, 0]).wait()
        pltpu.make_async_copy(w1b_hbm.at[e], w1b_v.at[slot],
                              sems.at[slot, 1]).wait()
        pltpu.make_async_copy(w2_hbm.at[e], w2_v.at[slot],
                              sems.at[slot, 2]).wait()

    @pl.when(j < used)
    def _():
        xs = xs_ref[...]
        a = jax.lax.dot_general(
            xs, w1a_v[slot], (((1,), (1,)), ((), ())),
            preferred_element_type=jnp.float32) + w1ab_ref[0]
        b = jax.lax.dot_general(
            xs, w1b_v[slot], (((1,), (1,)), ((), ())),
            preferred_element_type=jnp.float32) + w1bb_ref[0]
        h = (a / (1.0 + jnp.exp(-a))) * b
        y = jax.lax.dot_general(
            h, w2_v[slot], (((1,), (1,)), ((), ())),
            preferred_element_type=jnp.float32)
        ys_ref[...] = y + w2b_ref[0]


def _make_combine(P, S, D, PMAX, TPW):
    mesh = plsc.VectorSubcoreMesh(core_axis_name="c", subcore_axis_name="s")
    NCH = TPW // 16

    @functools.partial(
        pl.kernel, mesh=mesh,
        out_type=jax.ShapeDtypeStruct((S, D), jnp.float32),
        scratch_types=[
            pltpu.VMEM((NCH, 32), jnp.int32),
            pltpu.VMEM((2 * TPW,), jnp.float32),
            pltpu.VMEM((32, D), jnp.float32),
            pltpu.VMEM((16, D), jnp.float32),
            pltpu.SemaphoreType.DMA,
        ],
    )
    def combine_kernel(ys_hbm, dst_hbm, gw_hbm, out_hbm, dbuf, gbuf, rows,
                       obuf, sem):
        w = lax.axis_index("s") * 2 + lax.axis_index("c")
        for c in range(NCH):
            pltpu.sync_copy(dst_hbm.at[pl.ds(w * 2 * TPW + c * 32, 32)],
                            dbuf.at[c])
        pltpu.sync_copy(gw_hbm.at[pl.ds(w * 2 * TPW, 2 * TPW)], gbuf)
        for c in range(NCH):
            pltpu.async_copy(ys_hbm.at[dbuf.at[c]], rows, sem).wait()
            gv0 = gbuf[pl.ds(c * 32, 16)]
            gv1 = gbuf[pl.ds(c * 32 + 16, 16)]
            ws = []
            for t in range(16):
                gv = gv0 if t < 8 else gv1
                ws.append((gv[(2 * t) % 16], gv[(2 * t + 1) % 16]))

            def body(d, _):
                off = pl.multiple_of(d * 16, 16)
                for t in range(16):
                    w0, w1 = ws[t]
                    obuf[t, pl.ds(off, 16)] = (
                        w0 * rows[2 * t, pl.ds(off, 16)]
                        + w1 * rows[2 * t + 1, pl.ds(off, 16)])
                return 0

            lax.fori_loop(0, D // 16, body, 0)
            pltpu.sync_copy(obuf, out_hbm.at[pl.ds(w * TPW + c * 16, 16)])

    return combine_kernel


def kernel(x, ln_w, Wg, bg, w1a_W, w1a_b, w1b_W, w1b_b, w2_W, w2_b):
    B, S, D = x.shape
    E, F = w1a_W.shape[0], w1a_W.shape[1]
    K = 2
    P = S * K
    NB = P // TB + E + 1
    PMAX = NB * TB
    TPW = S // NW
    xf = x.reshape(S, D)

    xn, dst2, gw2, meta2 = pl.pallas_call(
        functools.partial(_router_body, n_experts=E, nb=NB),
        grid=(1,),
        in_specs=[
            pl.BlockSpec((S, D), lambda i: (0, 0)),
            pl.BlockSpec((1, D), lambda i: (0, 0)),
            pl.BlockSpec((E, D), lambda i: (0, 0)),
            pl.BlockSpec((1, E), lambda i: (0, 0)),
        ],
        out_specs=[
            pl.BlockSpec((S, D), lambda i: (0, 0)),
            pl.BlockSpec((S, K), lambda i: (0, 0)),
            pl.BlockSpec((S, K), lambda i: (0, 0)),
            pl.BlockSpec((1, 32), lambda i: (0, 0)),
        ],
        out_shape=[
            jax.ShapeDtypeStruct((S, D), jnp.float32),
            jax.ShapeDtypeStruct((S, K), jnp.int32),
            jax.ShapeDtypeStruct((S, K), jnp.float32),
            jax.ShapeDtypeStruct((1, 32), jnp.int32),
        ],
        compiler_params=pltpu.CompilerParams(
            vmem_limit_bytes=128 * 1024 * 1024,
        ),
    )(xf, ln_w.reshape(1, D), Wg, bg.reshape(1, E))

    dst = dst2.reshape(P)
    gws = gw2.reshape(P)
    meta = meta2.reshape(32)

    tok = jnp.arange(P, dtype=jnp.int32) // 2
    xs = _make_dispatch(P, S, D, PMAX)(tok, dst, xn)

    ys = pl.pallas_call(
        functools.partial(_ffn_body, nb=NB),
        grid_spec=pltpu.PrefetchScalarGridSpec(
            num_scalar_prefetch=1,
            grid=(NB,),
            in_specs=[
                pl.BlockSpec((TB, D), lambda j, m: (j, 0)),
                pl.BlockSpec(memory_space=pl.ANY),
                pl.BlockSpec((1, 1, F), lambda j, m: (jax.lax.rem(m[j], 8), 0, 0)),
                pl.BlockSpec(memory_space=pl.ANY),
                pl.BlockSpec((1, 1, F), lambda j, m: (jax.lax.rem(m[j], 8), 0, 0)),
                pl.BlockSpec(memory_space=pl.ANY),
                pl.BlockSpec((1, 1, D), lambda j, m: (jax.lax.rem(m[j], 8), 0, 0)),
            ],
            out_specs=pl.BlockSpec((TB, D), lambda j, m: (j, 0)),
            scratch_shapes=[
                pltpu.VMEM((2, F, D), jnp.float32),
                pltpu.VMEM((2, F, D), jnp.float32),
                pltpu.VMEM((2, D, F), jnp.float32),
                pltpu.SemaphoreType.DMA((2, 3)),
            ],
        ),
        out_shape=jax.ShapeDtypeStruct((PMAX, D), jnp.float32),
        compiler_params=pltpu.CompilerParams(
            dimension_semantics=("arbitrary",),
            vmem_limit_bytes=128 * 1024 * 1024,
        ),
    )(meta, xs, w1a_W, w1a_b.reshape(E, 1, F), w1b_W,
      w1b_b.reshape(E, 1, F), w2_W, w2_b.reshape(E, 1, D))

    out = _make_combine(P, S, D, PMAX, TPW)(ys, dst, gws)
    return out.reshape(B, S, D)


# speedup vs baseline: 1.6668x; 1.0349x over previous
import functools

import jax
import jax.numpy as jnp
import numpy as np
from jax import lax
from jax.experimental import pallas as pl
from jax.experimental.pallas import tpu as pltpu
from jax.experimental.pallas import tpu_sc as plsc

_EPS = float(np.finfo(np.float32).eps)

NW = 32
TB = 512


def _router_body(x_ref, lnw_ref, wg_ref, bg_ref, xn_ref, dst_ref, gw_ref,
                 meta_ref, *, n_experts, nb):
    E = n_experts
    S = x_ref.shape[0]
    x = x_ref[...]
    ms = jnp.mean(x * x, axis=-1, keepdims=True)
    xn = x * jax.lax.rsqrt(ms + _EPS) * lnw_ref[...]
    xn_ref[...] = xn
    logits = jax.lax.dot_general(
        xn, wg_ref[...], (((1,), (1,)), ((), ())),
        preferred_element_type=jnp.float32) + bg_ref[...]
    iota = jax.lax.broadcasted_iota(jnp.int32, logits.shape, 1)
    m1 = jnp.max(logits, axis=-1, keepdims=True)
    i1 = jnp.min(jnp.where(logits == m1, iota, E), axis=-1, keepdims=True)
    l2 = jnp.where(iota == i1, -jnp.inf, logits)
    m2 = jnp.max(l2, axis=-1, keepdims=True)
    i2 = jnp.min(jnp.where(l2 == m2, iota, E), axis=-1, keepdims=True)
    p = jnp.exp(m2 - m1)
    g1 = 1.0 / (1.0 + p)
    gw_ref[...] = jnp.concatenate([g1, p * g1], axis=1)

    oh1 = (iota == i1).astype(jnp.float32)
    oh2 = (iota == i2).astype(jnp.float32)
    ohs = oh1 + oh2
    ri = jax.lax.broadcasted_iota(jnp.int32, (S, S), 0)
    ci = jax.lax.broadcasted_iota(jnp.int32, (S, S), 1)
    tri = jnp.where(ri > ci, 1.0, 0.0)
    cum_ex = jax.lax.dot_general(tri, ohs, (((1,), (0,)), ((), ())),
                                 preferred_element_type=jnp.float32)

    offs = []
    ends_blk = []
    starts_blk = []
    present = []
    run = jnp.int32(0)
    for e in range(E):
        g_e = jnp.sum(jnp.where(iota == e, ohs, 0.0)).astype(jnp.int32)
        offs.append(run)
        present.append(g_e > 0)
        starts_blk.append(run // TB)
        nblk_e = (g_e + (TB - 1)) // TB
        run = run + nblk_e * TB
        ends_blk.append(run // TB)
    used = run // TB

    off_vec = jnp.zeros((1, E), jnp.float32)
    for e in range(E):
        off_vec = jnp.where(iota[:1] == e, offs[e].astype(jnp.float32),
                            off_vec)
    base = off_vec + cum_ex
    d1 = jnp.sum(oh1 * base, axis=1, keepdims=True)
    d2 = jnp.sum(oh2 * (base + oh1), axis=1, keepdims=True)
    dst_ref[...] = jnp.concatenate([d1, d2], axis=1).astype(jnp.int32)

    iota32 = jax.lax.broadcasted_iota(jnp.int32, (1, 32), 1)
    bid = jnp.zeros((1, 32), jnp.int32)
    rank = jnp.zeros((1, 32), jnp.int32)
    bid_last = jnp.int32(0)
    for e in range(E):
        bid = bid + jnp.where(iota32 >= ends_blk[e], 1, 0)
        bid_last = bid_last + jnp.where(ends_blk[e] <= used - 1, 1, 0)
        rank = rank + jnp.where((iota32 >= starts_blk[e]) & present[e], 1, 0)
    bid = jnp.where(iota32 < used, bid, bid_last)
    slot = jax.lax.rem(rank - 1, 2)
    meta_ref[...] = jnp.where(iota32 == nb, used, bid + 8 * slot)


def _make_dispatch(P, S, D, PMAX):
    PPW = P // NW
    NCH = PPW // 32
    mesh = plsc.VectorSubcoreMesh(core_axis_name="c", subcore_axis_name="s")

    @functools.partial(
        pl.kernel, mesh=mesh,
        out_type=jax.ShapeDtypeStruct((PMAX, D), jnp.float32),
        scratch_types=[
            pltpu.VMEM((NCH, 32), jnp.int32),
            pltpu.VMEM((NCH, 32), jnp.int32),
            pltpu.VMEM((2, 32, D), jnp.float32),
            pltpu.SemaphoreType.DMA,
            pltpu.SemaphoreType.DMA,
            pltpu.SemaphoreType.DMA,
            pltpu.SemaphoreType.DMA,
        ],
    )
    def dispatch_kernel(tok_hbm, dst_hbm, xn_hbm, xs_hbm, dstb, tokb, rows,
                        sg0, sg1, ss0, ss1):
        w = lax.axis_index("s") * 2 + lax.axis_index("c")
        sg = [sg0, sg1]
        ss = [ss0, ss1]
        for c in range(NCH):
            pltpu.sync_copy(dst_hbm.at[pl.ds(w * PPW + c * 32, 32)],
                            dstb.at[c])
            pltpu.sync_copy(tok_hbm.at[pl.ds(w * PPW + c * 32, 32)],
                            tokb.at[c])
        gh = [None] * NCH
        sh = [None] * NCH
        gh[0] = pltpu.async_copy(xn_hbm.at[tokb.at[0]], rows.at[0], sg[0])
        for c in range(NCH):
            gh[c].wait()
            sh[c] = pltpu.async_copy(rows.at[c % 2], xs_hbm.at[dstb.at[c]],
                                     ss[c % 2])
            if c + 1 < NCH:
                if c - 1 >= 0:
                    sh[c - 1].wait()
                gh[c + 1] = pltpu.async_copy(
                    xn_hbm.at[tokb.at[c + 1]], rows.at[(c + 1) % 2],
                    sg[(c + 1) % 2])
        if NCH >= 2:
            sh[NCH - 2].wait()
        sh[NCH - 1].wait()

    return dispatch_kernel


def _ffn_body(m_ref, xs_ref, w1a_ref, w1ab_ref, w1b_ref, w1bb_ref, w2_ref,
              w2b_ref, ys_ref, *, nb):
    j = pl.program_id(0)
    used = m_ref[nb]

    @pl.when(j < used)
    def _():
        xs = xs_ref[...]
        a = jax.lax.dot_general(
            xs, w1a_ref[0], (((1,), (1,)), ((), ())),
            preferred_element_type=jnp.float32) + w1ab_ref[0]
        b = jax.lax.dot_general(
            xs, w1b_ref[0], (((1,), (1,)), ((), ())),
            preferred_element_type=jnp.float32) + w1bb_ref[0]
        h = (a / (1.0 + jnp.exp(-a))) * b
        y = jax.lax.dot_general(
            h, w2_ref[0], (((1,), (1,)), ((), ())),
            preferred_element_type=jnp.float32)
        ys_ref[...] = y + w2b_ref[0]


def _make_combine(P, S, D, PMAX, TPW):
    mesh = plsc.VectorSubcoreMesh(core_axis_name="c", subcore_axis_name="s")
    NCH = TPW // 16

    @functools.partial(
        pl.kernel, mesh=mesh,
        out_type=jax.ShapeDtypeStruct((S, D), jnp.float32),
        scratch_types=[
            pltpu.VMEM((NCH, 32), jnp.int32),
            pltpu.VMEM((2 * TPW,), jnp.float32),
            pltpu.VMEM((2, 32, D), jnp.float32),
            pltpu.VMEM((16, D), jnp.float32),
            pltpu.SemaphoreType.DMA,
            pltpu.SemaphoreType.DMA,
        ],
    )
    def combine_kernel(ys_hbm, dst_hbm, gw_hbm, out_hbm, dbuf, gbuf, rows,
                       obuf, sem0, sem1):
        w = lax.axis_index("s") * 2 + lax.axis_index("c")
        sg = [sem0, sem1]
        for c in range(NCH):
            pltpu.sync_copy(dst_hbm.at[pl.ds(w * 2 * TPW + c * 32, 32)],
                            dbuf.at[c])
        pltpu.sync_copy(gw_hbm.at[pl.ds(w * 2 * TPW, 2 * TPW)], gbuf)
        gh = [None] * NCH
        gh[0] = pltpu.async_copy(ys_hbm.at[dbuf.at[0]], rows.at[0], sg[0])
        for c in range(NCH):
            gh[c].wait()
            if c + 1 < NCH:
                gh[c + 1] = pltpu.async_copy(
                    ys_hbm.at[dbuf.at[c + 1]], rows.at[(c + 1) % 2],
                    sg[(c + 1) % 2])
            gv0 = gbuf[pl.ds(c * 32, 16)]
            gv1 = gbuf[pl.ds(c * 32 + 16, 16)]
            ws = []
            for t in range(16):
                gv = gv0 if t < 8 else gv1
                ws.append((gv[(2 * t) % 16], gv[(2 * t + 1) % 16]))
            rb = rows.at[c % 2]

            def body(d, _):
                off = pl.multiple_of(d * 16, 16)
                for t in range(16):
                    w0, w1 = ws[t]
                    obuf[t, pl.ds(off, 16)] = (
                        w0 * rb[2 * t, pl.ds(off, 16)]
                        + w1 * rb[2 * t + 1, pl.ds(off, 16)])
                return 0

            lax.fori_loop(0, D // 16, body, 0)
            pltpu.sync_copy(obuf, out_hbm.at[pl.ds(w * TPW + c * 16, 16)])

    return combine_kernel


def kernel(x, ln_w, Wg, bg, w1a_W, w1a_b, w1b_W, w1b_b, w2_W, w2_b):
    B, S, D = x.shape
    E, F = w1a_W.shape[0], w1a_W.shape[1]
    K = 2
    P = S * K
    NB = P // TB + E + 1
    PMAX = NB * TB
    TPW = S // NW
    xf = x.reshape(S, D)

    xn, dst2, gw2, meta2 = pl.pallas_call(
        functools.partial(_router_body, n_experts=E, nb=NB),
        grid=(1,),
        in_specs=[
            pl.BlockSpec((S, D), lambda i: (0, 0)),
            pl.BlockSpec((1, D), lambda i: (0, 0)),
            pl.BlockSpec((E, D), lambda i: (0, 0)),
            pl.BlockSpec((1, E), lambda i: (0, 0)),
        ],
        out_specs=[
            pl.BlockSpec((S, D), lambda i: (0, 0)),
            pl.BlockSpec((S, K), lambda i: (0, 0)),
            pl.BlockSpec((S, K), lambda i: (0, 0)),
            pl.BlockSpec((1, 32), lambda i: (0, 0)),
        ],
        out_shape=[
            jax.ShapeDtypeStruct((S, D), jnp.float32),
            jax.ShapeDtypeStruct((S, K), jnp.int32),
            jax.ShapeDtypeStruct((S, K), jnp.float32),
            jax.ShapeDtypeStruct((1, 32), jnp.int32),
        ],
        compiler_params=pltpu.CompilerParams(
            vmem_limit_bytes=128 * 1024 * 1024,
        ),
    )(xf, ln_w.reshape(1, D), Wg, bg.reshape(1, E))

    dst = dst2.reshape(P)
    gws = gw2.reshape(P)
    meta = meta2.reshape(32)

    tok = jnp.arange(P, dtype=jnp.int32) // 2
    xs = _make_dispatch(P, S, D, PMAX)(tok, dst, xn)

    ys = pl.pallas_call(
        functools.partial(_ffn_body, nb=NB),
        grid_spec=pltpu.PrefetchScalarGridSpec(
            num_scalar_prefetch=1,
            grid=(NB,),
            in_specs=[
                pl.BlockSpec((TB, D), lambda j, m: (j, 0)),
                pl.BlockSpec((1, F, D), lambda j, m: (jax.lax.rem(m[j], 8), 0, 0)),
                pl.BlockSpec((1, 1, F), lambda j, m: (jax.lax.rem(m[j], 8), 0, 0)),
                pl.BlockSpec((1, F, D), lambda j, m: (jax.lax.rem(m[j], 8), 0, 0)),
                pl.BlockSpec((1, 1, F), lambda j, m: (jax.lax.rem(m[j], 8), 0, 0)),
                pl.BlockSpec((1, D, F), lambda j, m: (jax.lax.rem(m[j], 8), 0, 0)),
                pl.BlockSpec((1, 1, D), lambda j, m: (jax.lax.rem(m[j], 8), 0, 0)),
            ],
            out_specs=pl.BlockSpec((TB, D), lambda j, m: (j, 0)),
        ),
        out_shape=jax.ShapeDtypeStruct((PMAX, D), jnp.float32),
        compiler_params=pltpu.CompilerParams(
            dimension_semantics=("arbitrary",),
            vmem_limit_bytes=128 * 1024 * 1024,
        ),
    )(meta, xs, w1a_W, w1a_b.reshape(E, 1, F), w1b_W,
      w1b_b.reshape(E, 1, F), w2_W, w2_b.reshape(E, 1, D))

    out = _make_combine(P, S, D, PMAX, TPW)(ys, dst, gws)
    return out.reshape(B, S, D)
